# baseline (speedup 1.0000x reference)
import functools

import jax
import jax.numpy as jnp
from jax import lax
from jax.experimental import pallas as pl
from jax.experimental.pallas import tpu as pltpu
from jax.experimental.pallas import tpu_sc as plsc

N = 10000
E = 320000
H = 4
DE_L = 2 * H
DN_L = 4 * H

ROW = 125
ROWS_PER_W = E // (32 * ROW)
EDGES_PER_W = ROW * ROWS_PER_W
NROWS = E // ROW
NB_SC = 80

BE = 4000
BN = 2000

def _mesh():
    return plsc.VectorSubcoreMesh(
        core_axis_name="c", subcore_axis_name="s", num_cores=2, num_subcores=16
    )

def _sc_gather(table, idx2d):
    @functools.partial(
        pl.kernel,
        mesh=_mesh(),
        compiler_params=pltpu.CompilerParams(use_tc_tiling_on_sc=False),
        out_type=jax.ShapeDtypeStruct((E, DE_L), jnp.float32),
        scratch_types=[
            pltpu.VMEM((ROWS_PER_W, ROW), jnp.int32),
            pltpu.VMEM((EDGES_PER_W, DE_L), jnp.float32),
            pltpu.VMEM_SHARED((N, DE_L), jnp.float32),
            pltpu.SemaphoreType.DMA,
        ],
    )
    def k(tab_hbm, idx_hbm, g_hbm, idxb, rowsb, shared, sem):
        cid = lax.axis_index("c")
        sid = lax.axis_index("s")
        wid = sid * 2 + cid
        t0 = pltpu.async_copy(
            tab_hbm.at[pl.ds(sid * 625, 625)], shared.at[pl.ds(sid * 625, 625)], sem
        )
        a = pltpu.async_copy(
            idx_hbm.at[pl.ds(wid * ROWS_PER_W, ROWS_PER_W)], idxb, sem
        )
        t0.wait()
        a.wait()
        plsc.subcore_barrier()

        @pl.loop(0, ROWS_PER_W, step=NB_SC)
        def _(t):
            cps = [
                pltpu.async_copy(
                    shared.at[idxb.at[t + b]],
                    rowsb.at[pl.ds((t + b) * ROW, ROW)],
                    sem,
                )
                for b in range(NB_SC)
            ]
            for c in cps:
                c.wait()

        pltpu.sync_copy(rowsb, g_hbm.at[pl.ds(wid * EDGES_PER_W, EDGES_PER_W)])

    return k(table, idx2d)


def _sc_scatter(vals, idx2d, zer):
    @functools.partial(
        pl.kernel,
        mesh=_mesh(),
        compiler_params=pltpu.CompilerParams(use_tc_tiling_on_sc=False),
        out_type=jax.ShapeDtypeStruct((2, N, DE_L), jnp.float32),
        scratch_types=[
            pltpu.VMEM((ROWS_PER_W, ROW), jnp.int32),
            pltpu.VMEM((EDGES_PER_W, DE_L), jnp.float32),
            pltpu.VMEM_SHARED((N, DE_L), jnp.float32),
            pltpu.SemaphoreType.DMA,
        ],
    )
    def k(vals_hbm, idx_hbm, zer_hbm, out_hbm, idxb, valsb, shared, sem):
        cid = lax.axis_index("c")
        sid = lax.axis_index("s")
        base = cid * 16 + sid
        z0 = pltpu.async_copy(
            zer_hbm.at[pl.ds(sid * 624, 624)], shared.at[pl.ds(sid * 624, 624)], sem
        )
        z1 = pltpu.async_copy(
            zer_hbm.at[pl.ds(9360, 640)], shared.at[pl.ds(9360, 640)], sem
        )
        a = pltpu.async_copy(
            idx_hbm.at[pl.ds(base * ROWS_PER_W, ROWS_PER_W)], idxb, sem
        )
        b = pltpu.async_copy(
            vals_hbm.at[pl.ds(base * EDGES_PER_W, EDGES_PER_W)], valsb, sem
        )
        z0.wait()
        z1.wait()
        a.wait()
        b.wait()
        plsc.subcore_barrier()

        @pl.loop(0, ROWS_PER_W, step=NB_SC)
        def _(t):
            cps = [
                pltpu.async_copy(
                    valsb.at[pl.ds((t + b2) * ROW, ROW)],
                    shared.at[idxb.at[t + b2]],
                    sem,
                    add=True,
                )
                for b2 in range(NB_SC)
            ]
            for c in cps:
                c.wait()

        plsc.subcore_barrier()
        pltpu.sync_copy(
            shared.at[pl.ds(sid * 624, 624)],
            out_hbm.at[cid, pl.ds(sid * 624, 624)],
        )

        @pl.when(sid == 15)
        def _():
            pltpu.sync_copy(
                shared.at[pl.ds(9360, 640)], out_hbm.at[cid, pl.ds(9360, 640)]
            )

    return k(vals, idx2d, zer)


def _sc_count(idx2d, ones, zer):
    @functools.partial(
        pl.kernel,
        mesh=_mesh(),
        compiler_params=pltpu.CompilerParams(use_tc_tiling_on_sc=False),
        out_type=jax.ShapeDtypeStruct((2, N, DE_L), jnp.float32),
        scratch_types=[
            pltpu.VMEM((ROWS_PER_W, ROW), jnp.int32),
            pltpu.VMEM((ROW, DE_L), jnp.float32),
            pltpu.VMEM_SHARED((N, DE_L), jnp.float32),
            pltpu.SemaphoreType.DMA,
        ],
    )
    def k(idx_hbm, ones_hbm, zer_hbm, out_hbm, idxb, onesb, shared, sem):
        cid = lax.axis_index("c")
        sid = lax.axis_index("s")
        base = cid * 16 + sid
        z0 = pltpu.async_copy(
            zer_hbm.at[pl.ds(sid * 624, 624)], shared.at[pl.ds(sid * 624, 624)], sem
        )
        z1 = pltpu.async_copy(
            zer_hbm.at[pl.ds(9360, 640)], shared.at[pl.ds(9360, 640)], sem
        )
        a = pltpu.async_copy(
            idx_hbm.at[pl.ds(base * ROWS_PER_W, ROWS_PER_W)], idxb, sem
        )
        b = pltpu.async_copy(ones_hbm, onesb, sem)
        z0.wait()
        z1.wait()
        a.wait()
        b.wait()
        plsc.subcore_barrier()

        @pl.loop(0, ROWS_PER_W, step=NB_SC)
        def _(t):
            cps = [
                pltpu.async_copy(onesb, shared.at[idxb.at[t + b2]], sem, add=True)
                for b2 in range(NB_SC)
            ]
            for c in cps:
                c.wait()

        plsc.subcore_barrier()
        pltpu.sync_copy(
            shared.at[pl.ds(sid * 624, 624)],
            out_hbm.at[cid, pl.ds(sid * 624, 624)],
        )

        @pl.when(sid == 15)
        def _():
            pltpu.sync_copy(
                shared.at[pl.ds(9360, 640)], out_hbm.at[cid, pl.ds(9360, 640)]
            )

    return k(idx2d, ones, zer)


BC = 64000


def _edge_encoder(eat, W1t, b1c, W2t, b2c):
    def body(ea, w1, c1, w2, c2, o):
        m = jnp.maximum(
            jnp.dot(w1[...], ea[...], preferred_element_type=jnp.float32) + c1[...],
            0.0,
        )
        o[...] = jnp.maximum(
            jnp.dot(w2[...], m, preferred_element_type=jnp.float32) + c2[...], 0.0
        )

    return pl.pallas_call(
        body,
        grid=(E // BC,),
        in_specs=[
            pl.BlockSpec((16, BC), lambda i: (0, i)),
            pl.BlockSpec((4, 16), lambda i: (0, 0)),
            pl.BlockSpec((4, 1), lambda i: (0, 0)),
            pl.BlockSpec((8, 4), lambda i: (0, 0)),
            pl.BlockSpec((8, 1), lambda i: (0, 0)),
        ],
        out_specs=pl.BlockSpec((8, BC), lambda i: (0, i)),
        out_shape=jax.ShapeDtypeStruct((8, E), jnp.float32),
    )(eat, W1t, b1c, W2t, b2c)


def _node_encoder(x, Wn1, bn1, Wn2, bn2, Whe_s, Wg2, bg2, Wge_rep, bhe_rep):
    nsteps = N // BN

    def body(xb, w1, c1, w2, c2, ws, wg, cg, wgr, br, n0, s1, u0, ct, acc):
        i = pl.program_id(0)

        @pl.when(i == 0)
        def _():
            acc[...] = jnp.zeros_like(acc)

        m = jnp.maximum(
            jnp.dot(xb[...], w1[...], preferred_element_type=jnp.float32) + c1[...],
            0.0,
        )
        n2 = jnp.maximum(
            jnp.dot(m, w2[...], preferred_element_type=jnp.float32) + c2[...], 0.0
        )
        n0[...] = n2
        s1[...] = jnp.dot(n2, ws[...], preferred_element_type=jnp.float32)
        acc[...] += jnp.sum(n2, axis=0, keepdims=True)

        @pl.when(i == nsteps - 1)
        def _():
            u = (
                jnp.dot(acc[...] / N, wg[...], preferred_element_type=jnp.float32)
                + cg[...]
            )
            u0[...] = u
            ct[...] = (
                jnp.dot(u, wgr[...], preferred_element_type=jnp.float32) + br[...]
            )

    return pl.pallas_call(
        body,
        grid=(nsteps,),
        in_specs=[
            pl.BlockSpec((BN, 128), lambda i: (i, 0)),
            pl.BlockSpec((128, 32), lambda i: (0, 0)),
            pl.BlockSpec((1, 32), lambda i: (0, 0)),
            pl.BlockSpec((32, 16), lambda i: (0, 0)),
            pl.BlockSpec((1, 16), lambda i: (0, 0)),
            pl.BlockSpec((16, 8), lambda i: (0, 0)),
            pl.BlockSpec((16, 4), lambda i: (0, 0)),
            pl.BlockSpec((1, 4), lambda i: (0, 0)),
            pl.BlockSpec((4, 128), lambda i: (0, 0)),
            pl.BlockSpec((1, 128), lambda i: (0, 0)),
        ],
        out_specs=[
            pl.BlockSpec((BN, 16), lambda i: (i, 0)),
            pl.BlockSpec((BN, 8), lambda i: (i, 0)),
            pl.BlockSpec((1, 4), lambda i: (0, 0)),
            pl.BlockSpec((1, 128), lambda i: (0, 0)),
        ],
        out_shape=[
            jax.ShapeDtypeStruct((N, 16), jnp.float32),
            jax.ShapeDtypeStruct((N, 8), jnp.float32),
            jax.ShapeDtypeStruct((1, 4), jnp.float32),
            jax.ShapeDtypeStruct((1, 128), jnp.float32),
        ],
        scratch_shapes=[pltpu.VMEM((1, 16), jnp.float32)],
    )(x, Wn1, bn1, Wn2, bn2, Whe_s, Wg2, bg2, Wge_rep, bhe_rep)


def _edge_hop(epk, gpk, ct, Wbd, write_enew):
    nrows = epk.shape[0]
    nsteps = nrows // BE

    def body(e, g_hbm, c, w, *refs):
        if write_enew:
            enew, eres, sume, acc, gbuf, gsem = refs
        else:
            eres, sume, acc, gbuf, gsem = refs
        i = pl.program_id(0)

        @pl.when(i == 0)
        def _():
            acc[...] = jnp.zeros_like(acc)
            pltpu.make_async_copy(
                g_hbm.at[pl.ds(0, BE)], gbuf.at[0], gsem.at[0]
            ).start()

        @pl.when(i + 1 < nsteps)
        def _():
            pltpu.make_async_copy(
                g_hbm.at[pl.ds((i + 1) * BE, BE)],
                gbuf.at[(i + 1) % 2],
                gsem.at[(i + 1) % 2],
            ).start()

        pltpu.make_async_copy(
            g_hbm.at[pl.ds(i * BE, BE)], gbuf.at[i % 2], gsem.at[i % 2]
        ).wait()
        er = jnp.maximum(
            jnp.dot(e[...], w[...], preferred_element_type=jnp.float32)
            + gbuf[i % 2]
            + c[...],
            0.0,
        )
        eres[...] = er
        if write_enew:
            enew[...] = e[...] + er
        acc[...] += jnp.sum(er, axis=0, keepdims=True)

        @pl.when(i == nsteps - 1)
        def _():
            sume[...] = acc[...]

    out_specs = [
        pl.BlockSpec((BE, 128), lambda i: (i, 0)),
        pl.BlockSpec((1, 128), lambda i: (0, 0)),
    ]
    out_shape = [
        jax.ShapeDtypeStruct((nrows, 128), jnp.float32),
        jax.ShapeDtypeStruct((1, 128), jnp.float32),
    ]
    if write_enew:
        out_specs = [pl.BlockSpec((BE, 128), lambda i: (i, 0))] + out_specs
        out_shape = [jax.ShapeDtypeStruct((nrows, 128), jnp.float32)] + out_shape

    return pl.pallas_call(
        body,
        grid=(nsteps,),
        in_specs=[
            pl.BlockSpec((BE, 128), lambda i: (i, 0)),
            pl.BlockSpec(memory_space=pltpu.MemorySpace.HBM),
            pl.BlockSpec((1, 128), lambda i: (0, 0)),
            pl.BlockSpec((128, 128), lambda i: (0, 0)),
        ],
        out_specs=out_specs,
        out_shape=out_shape,
        scratch_shapes=[
            pltpu.VMEM((1, 128), jnp.float32),
            pltpu.VMEM((2, BE, 128), jnp.float32),
            pltpu.SemaphoreType.DMA((2,)),
        ],
        input_output_aliases={1: 1 if write_enew else 0},
    )(epk, gpk, ct, Wbd)


def _node_hop(n, aggA, aggB, cntA, cntB, u, sume, weights, last_hop):
    (Whn_n, Whn_in, Whn_g, bhn, Wge_fold, Whg_n, Whg_g, bhg,
     Whe_s, Wge_rep, bhe_rep, Wro_n, bro_n, Wro_g, bro_g) = weights
    nsteps = N // BN

    def body(nb, aA, aB, cA, cB, ub, se, wnn, wni, wng, cbn, wgef, wgn, wgg,
             cbg, ws, wgr, cbr, wron, bron, wrog, brog, *refs):
        if last_hop:
            nout, gout, acc = refs
        else:
            nnew_r, snext_r, unew_r, cnext_r, acc = refs
        i = pl.program_id(0)

        @pl.when(i == 0)
        def _():
            acc[...] = jnp.zeros_like(acc)

        d = jnp.dot(ub[...], wng[...], preferred_element_type=jnp.float32) + cbn[...]
        agg = (aA[...] + aB[...]) / jnp.maximum(cA[...] + cB[...], 1.0)
        nr = jnp.maximum(
            jnp.dot(nb[...], wnn[...], preferred_element_type=jnp.float32)
            + jnp.dot(agg, wni[...], preferred_element_type=jnp.float32)
            + d,
            0.0,
        )
        nnew = nb[...] + nr
        acc[...] += jnp.sum(nr, axis=0, keepdims=True)
        if last_hop:
            nout[...] = jax.nn.sigmoid(
                jnp.dot(nnew, wron[...], preferred_element_type=jnp.float32)
                + bron[...]
            )
        else:
            nnew_r[...] = nnew
            snext_r[...] = jnp.dot(nnew, ws[...], preferred_element_type=jnp.float32)

        @pl.when(i == nsteps - 1)
        def _():
            ures = jnp.maximum(
                jnp.dot(se[...], wgef[...], preferred_element_type=jnp.float32)
                + jnp.dot(acc[...] / N, wgn[...], preferred_element_type=jnp.float32)
                + jnp.dot(ub[...], wgg[...], preferred_element_type=jnp.float32)
                + cbg[...],
                0.0,
            )
            unew = ub[...] + ures
            if last_hop:
                gout[...] = jax.nn.sigmoid(
                    jnp.dot(unew, wrog[...], preferred_element_type=jnp.float32)
                    + brog[...]
                )
            else:
                unew_r[...] = unew
                cnext_r[...] = (
                    jnp.dot(unew, wgr[...], preferred_element_type=jnp.float32)
                    + cbr[...]
                )

    if last_hop:
        out_specs = [
            pl.BlockSpec((BN, 1), lambda i: (i, 0)),
            pl.BlockSpec((1, 1), lambda i: (0, 0)),
        ]
        out_shape = [
            jax.ShapeDtypeStruct((N, 1), jnp.float32),
            jax.ShapeDtypeStruct((1, 1), jnp.float32),
        ]
    else:
        out_specs = [
            pl.BlockSpec((BN, 16), lambda i: (i, 0)),
            pl.BlockSpec((BN, 8), lambda i: (i, 0)),
            pl.BlockSpec((1, 4), lambda i: (0, 0)),
            pl.BlockSpec((1, 128), lambda i: (0, 0)),
        ]
        out_shape = [
            jax.ShapeDtypeStruct((N, 16), jnp.float32),
            jax.ShapeDtypeStruct((N, 8), jnp.float32),
            jax.ShapeDtypeStruct((1, 4), jnp.float32),
            jax.ShapeDtypeStruct((1, 128), jnp.float32),
        ]

    return pl.pallas_call(
        body,
        grid=(nsteps,),
        in_specs=[
            pl.BlockSpec((BN, 16), lambda i: (i, 0)),
            pl.BlockSpec((BN, 8), lambda i: (i, 0)),
            pl.BlockSpec((BN, 8), lambda i: (i, 0)),
            pl.BlockSpec((BN, 8), lambda i: (i, 0)),
            pl.BlockSpec((BN, 8), lambda i: (i, 0)),
            pl.BlockSpec((1, 4), lambda i: (0, 0)),
            pl.BlockSpec((1, 128), lambda i: (0, 0)),
            pl.BlockSpec((16, 16), lambda i: (0, 0)),
            pl.BlockSpec((8, 16), lambda i: (0, 0)),
            pl.BlockSpec((4, 16), lambda i: (0, 0)),
            pl.BlockSpec((1, 16), lambda i: (0, 0)),
            pl.BlockSpec((128, 4), lambda i: (0, 0)),
            pl.BlockSpec((16, 4), lambda i: (0, 0)),
            pl.BlockSpec((4, 4), lambda i: (0, 0)),
            pl.BlockSpec((1, 4), lambda i: (0, 0)),
            pl.BlockSpec((16, 8), lambda i: (0, 0)),
            pl.BlockSpec((4, 128), lambda i: (0, 0)),
            pl.BlockSpec((1, 128), lambda i: (0, 0)),
            pl.BlockSpec((16, 1), lambda i: (0, 0)),
            pl.BlockSpec((1, 1), lambda i: (0, 0)),
            pl.BlockSpec((4, 1), lambda i: (0, 0)),
            pl.BlockSpec((1, 1), lambda i: (0, 0)),
        ],
        out_specs=out_specs,
        out_shape=out_shape,
        scratch_shapes=[pltpu.VMEM((1, 16), jnp.float32)],
    )(n, aggA, aggB, cntA, cntB, u, sume, Whn_n, Whn_in, Whn_g, bhn,
      Wge_fold, Whg_n, Whg_g, bhg, Whe_s, Wge_rep, bhe_rep, Wro_n, bro_n,
      Wro_g, bro_g)


def kernel(x, edge_attr, senders, receivers, We1, be1, Wn1, bn1, We2, be2,
           Wn2, bn2, Wg2, bg2, Whe_e, Whe_s, Whe_g, bhe, Whn_n, Whn_in,
           Whn_g, bhn, Whg_e, Whg_n, Whg_g, bhg, Wro_n, bro_n, Wro_g, bro_g):
    f32 = jnp.float32
    senders = senders.astype(jnp.int32)
    receivers = receivers.astype(jnp.int32)
    s2d = senders.reshape(NROWS, ROW)
    r2d = receivers.reshape(NROWS, ROW)

    eye16 = jnp.eye(16, dtype=f32)
    W1t = We1.T
    b1c = be1[:, None]
    W2t = We2.T
    b2c = be2[:, None]
    Wbd = jnp.kron(eye16, Whe_e)
    Wge_rep = jnp.tile(Whe_g, (1, 16))
    bhe_rep = jnp.tile(bhe, 16)[None]
    Wge_fold = jnp.tile(Whg_e, (16, 1)) / E
    bn1_2 = bn1[None]
    bn2_2 = bn2[None]
    bg2_2 = bg2[None]
    bhn_2 = bhn[None]
    bhg_2 = bhg[None]
    bro_n2 = bro_n[None]
    bro_g2 = bro_g[None]

    zer = jnp.zeros((N, DE_L), f32)
    ones = jnp.ones((ROW, DE_L), f32)

    e0t = _edge_encoder(edge_attr.T, W1t, b1c, W2t, b2c)
    e0pk = e0t.T.reshape(E // 16, 128)
    n0, s1, u0, c1 = _node_encoder(
        x, Wn1, bn1_2, Wn2, bn2_2, Whe_s, Wg2, bg2_2, Wge_rep, bhe_rep
    )
    cntP = _sc_count(r2d, ones, zer)
    cntA, cntB = cntP[0], cntP[1]

    node_w = (Whn_n, Whn_in, Whn_g, bhn_2, Wge_fold, Whg_n, Whg_g, bhg_2,
              Whe_s, Wge_rep, bhe_rep, Wro_n, bro_n2, Wro_g, bro_g2)

    g1 = _sc_gather(s1, s2d).reshape(E // 16, 128)
    e1pk, er1pk, sume1 = _edge_hop(e0pk, g1, c1, Wbd, write_enew=True)
    aggP1 = _sc_scatter(er1pk.reshape(E, DE_L), r2d, zer)
    n1, s2n, u1, c2 = _node_hop(
        n0, aggP1[0], aggP1[1], cntA, cntB, u0, sume1, node_w, last_hop=False
    )

    g2 = _sc_gather(s2n, s2d).reshape(E // 16, 128)
    er2pk, sume2 = _edge_hop(e1pk, g2, c2, Wbd, write_enew=False)
    aggP2 = _sc_scatter(er2pk.reshape(E, DE_L), r2d, zer)
    nodes_out, globals_out = _node_hop(
        n1, aggP2[0], aggP2[1], cntA, cntB, u1, sume2, node_w, last_hop=True
    )
    return nodes_out, globals_out


# speedup vs baseline: 1.0062x; 1.0062x over previous
import functools

import jax
import jax.numpy as jnp
from jax import lax
from jax.experimental import pallas as pl
from jax.experimental.pallas import tpu as pltpu
from jax.experimental.pallas import tpu_sc as plsc

N = 10000
E = 320000
H = 4
DE_L = 2 * H
DN_L = 4 * H

ROW = 125
ROWS_PER_W = E // (32 * ROW)
EDGES_PER_W = ROW * ROWS_PER_W
NROWS = E // ROW
NB_SC = 40

BE = 4000
BN = 2000

def _mesh():
    return plsc.VectorSubcoreMesh(
        core_axis_name="c", subcore_axis_name="s", num_cores=2, num_subcores=16
    )

def _sc_gather(table, idx2d):
    @functools.partial(
        pl.kernel,
        mesh=_mesh(),
        compiler_params=pltpu.CompilerParams(use_tc_tiling_on_sc=False),
        out_type=jax.ShapeDtypeStruct((E, DE_L), jnp.float32),
        scratch_types=[
            pltpu.VMEM((ROWS_PER_W, ROW), jnp.int32),
            pltpu.VMEM((EDGES_PER_W, DE_L), jnp.float32),
            pltpu.VMEM_SHARED((N, DE_L), jnp.float32),
            pltpu.SemaphoreType.DMA,
        ],
    )
    def k(tab_hbm, idx_hbm, g_hbm, idxb, rowsb, shared, sem):
        cid = lax.axis_index("c")
        sid = lax.axis_index("s")
        wid = sid * 2 + cid
        t0 = pltpu.async_copy(
            tab_hbm.at[pl.ds(sid * 625, 625)], shared.at[pl.ds(sid * 625, 625)], sem
        )
        a = pltpu.async_copy(
            idx_hbm.at[pl.ds(wid * ROWS_PER_W, ROWS_PER_W)], idxb, sem
        )
        t0.wait()
        a.wait()
        plsc.subcore_barrier()

        @pl.loop(0, ROWS_PER_W, step=NB_SC)
        def _(t):
            cps = [
                pltpu.async_copy(
                    shared.at[idxb.at[t + b]],
                    rowsb.at[pl.ds((t + b) * ROW, ROW)],
                    sem,
                )
                for b in range(NB_SC)
            ]
            for c in cps:
                c.wait()

        pltpu.sync_copy(rowsb, g_hbm.at[pl.ds(wid * EDGES_PER_W, EDGES_PER_W)])

    return k(table, idx2d)


def _sc_scatter(vals, idx2d, zer):
    @functools.partial(
        pl.kernel,
        mesh=_mesh(),
        compiler_params=pltpu.CompilerParams(use_tc_tiling_on_sc=False),
        out_type=jax.ShapeDtypeStruct((2, N, DE_L), jnp.float32),
        scratch_types=[
            pltpu.VMEM((ROWS_PER_W, ROW), jnp.int32),
            pltpu.VMEM((EDGES_PER_W, DE_L), jnp.float32),
            pltpu.VMEM_SHARED((N, DE_L), jnp.float32),
            pltpu.SemaphoreType.DMA,
        ],
    )
    def k(vals_hbm, idx_hbm, zer_hbm, out_hbm, idxb, valsb, shared, sem):
        cid = lax.axis_index("c")
        sid = lax.axis_index("s")
        base = cid * 16 + sid
        z0 = pltpu.async_copy(
            zer_hbm.at[pl.ds(sid * 624, 624)], shared.at[pl.ds(sid * 624, 624)], sem
        )
        z1 = pltpu.async_copy(
            zer_hbm.at[pl.ds(9360, 640)], shared.at[pl.ds(9360, 640)], sem
        )
        a = pltpu.async_copy(
            idx_hbm.at[pl.ds(base * ROWS_PER_W, ROWS_PER_W)], idxb, sem
        )
        b = pltpu.async_copy(
            vals_hbm.at[pl.ds(base * EDGES_PER_W, EDGES_PER_W)], valsb, sem
        )
        z0.wait()
        z1.wait()
        a.wait()
        b.wait()
        plsc.subcore_barrier()

        @pl.loop(0, ROWS_PER_W, step=NB_SC)
        def _(t):
            cps = [
                pltpu.async_copy(
                    valsb.at[pl.ds((t + b2) * ROW, ROW)],
                    shared.at[idxb.at[t + b2]],
                    sem,
                    add=True,
                )
                for b2 in range(NB_SC)
            ]
            for c in cps:
                c.wait()

        plsc.subcore_barrier()
        pltpu.sync_copy(
            shared.at[pl.ds(sid * 624, 624)],
            out_hbm.at[cid, pl.ds(sid * 624, 624)],
        )

        @pl.when(sid == 15)
        def _():
            pltpu.sync_copy(
                shared.at[pl.ds(9360, 640)], out_hbm.at[cid, pl.ds(9360, 640)]
            )

    return k(vals, idx2d, zer)


def _sc_count(idx2d, ones, zer):
    @functools.partial(
        pl.kernel,
        mesh=_mesh(),
        compiler_params=pltpu.CompilerParams(use_tc_tiling_on_sc=False),
        out_type=jax.ShapeDtypeStruct((2, N, DE_L), jnp.float32),
        scratch_types=[
            pltpu.VMEM((ROWS_PER_W, ROW), jnp.int32),
            pltpu.VMEM((ROW, DE_L), jnp.float32),
            pltpu.VMEM_SHARED((N, DE_L), jnp.float32),
            pltpu.SemaphoreType.DMA,
        ],
    )
    def k(idx_hbm, ones_hbm, zer_hbm, out_hbm, idxb, onesb, shared, sem):
        cid = lax.axis_index("c")
        sid = lax.axis_index("s")
        base = cid * 16 + sid
        z0 = pltpu.async_copy(
            zer_hbm.at[pl.ds(sid * 624, 624)], shared.at[pl.ds(sid * 624, 624)], sem
        )
        z1 = pltpu.async_copy(
            zer_hbm.at[pl.ds(9360, 640)], shared.at[pl.ds(9360, 640)], sem
        )
        a = pltpu.async_copy(
            idx_hbm.at[pl.ds(base * ROWS_PER_W, ROWS_PER_W)], idxb, sem
        )
        b = pltpu.async_copy(ones_hbm, onesb, sem)
        z0.wait()
        z1.wait()
        a.wait()
        b.wait()
        plsc.subcore_barrier()

        @pl.loop(0, ROWS_PER_W, step=NB_SC)
        def _(t):
            cps = [
                pltpu.async_copy(onesb, shared.at[idxb.at[t + b2]], sem, add=True)
                for b2 in range(NB_SC)
            ]
            for c in cps:
                c.wait()

        plsc.subcore_barrier()
        pltpu.sync_copy(
            shared.at[pl.ds(sid * 624, 624)],
            out_hbm.at[cid, pl.ds(sid * 624, 624)],
        )

        @pl.when(sid == 15)
        def _():
            pltpu.sync_copy(
                shared.at[pl.ds(9360, 640)], out_hbm.at[cid, pl.ds(9360, 640)]
            )

    return k(idx2d, ones, zer)


BC = 64000


def _edge_encoder(eat, W1t, b1c, W2t, b2c):
    def body(ea, w1, c1, w2, c2, o):
        m = jnp.maximum(
            jnp.dot(w1[...], ea[...], preferred_element_type=jnp.float32) + c1[...],
            0.0,
        )
        o[...] = jnp.maximum(
            jnp.dot(w2[...], m, preferred_element_type=jnp.float32) + c2[...], 0.0
        )

    return pl.pallas_call(
        body,
        grid=(E // BC,),
        in_specs=[
            pl.BlockSpec((16, BC), lambda i: (0, i)),
            pl.BlockSpec((4, 16), lambda i: (0, 0)),
            pl.BlockSpec((4, 1), lambda i: (0, 0)),
            pl.BlockSpec((8, 4), lambda i: (0, 0)),
            pl.BlockSpec((8, 1), lambda i: (0, 0)),
        ],
        out_specs=pl.BlockSpec((8, BC), lambda i: (0, i)),
        out_shape=jax.ShapeDtypeStruct((8, E), jnp.float32),
    )(eat, W1t, b1c, W2t, b2c)


def _node_encoder(x, Wn1, bn1, Wn2, bn2, Whe_s, Wg2, bg2, Wge_rep, bhe_rep):
    nsteps = N // BN

    def body(xb, w1, c1, w2, c2, ws, wg, cg, wgr, br, n0, s1, u0, ct, acc):
        i = pl.program_id(0)

        @pl.when(i == 0)
        def _():
            acc[...] = jnp.zeros_like(acc)

        m = jnp.maximum(
            jnp.dot(xb[...], w1[...], preferred_element_type=jnp.float32) + c1[...],
            0.0,
        )
        n2 = jnp.maximum(
            jnp.dot(m, w2[...], preferred_element_type=jnp.float32) + c2[...], 0.0
        )
        n0[...] = n2
        s1[...] = jnp.dot(n2, ws[...], preferred_element_type=jnp.float32)
        acc[...] += jnp.sum(n2, axis=0, keepdims=True)

        @pl.when(i == nsteps - 1)
        def _():
            u = (
                jnp.dot(acc[...] / N, wg[...], preferred_element_type=jnp.float32)
                + cg[...]
            )
            u0[...] = u
            ct[...] = (
                jnp.dot(u, wgr[...], preferred_element_type=jnp.float32) + br[...]
            )

    return pl.pallas_call(
        body,
        grid=(nsteps,),
        in_specs=[
            pl.BlockSpec((BN, 128), lambda i: (i, 0)),
            pl.BlockSpec((128, 32), lambda i: (0, 0)),
            pl.BlockSpec((1, 32), lambda i: (0, 0)),
            pl.BlockSpec((32, 16), lambda i: (0, 0)),
            pl.BlockSpec((1, 16), lambda i: (0, 0)),
            pl.BlockSpec((16, 8), lambda i: (0, 0)),
            pl.BlockSpec((16, 4), lambda i: (0, 0)),
            pl.BlockSpec((1, 4), lambda i: (0, 0)),
            pl.BlockSpec((4, 128), lambda i: (0, 0)),
            pl.BlockSpec((1, 128), lambda i: (0, 0)),
        ],
        out_specs=[
            pl.BlockSpec((BN, 16), lambda i: (i, 0)),
            pl.BlockSpec((BN, 8), lambda i: (i, 0)),
            pl.BlockSpec((1, 4), lambda i: (0, 0)),
            pl.BlockSpec((1, 128), lambda i: (0, 0)),
        ],
        out_shape=[
            jax.ShapeDtypeStruct((N, 16), jnp.float32),
            jax.ShapeDtypeStruct((N, 8), jnp.float32),
            jax.ShapeDtypeStruct((1, 4), jnp.float32),
            jax.ShapeDtypeStruct((1, 128), jnp.float32),
        ],
        scratch_shapes=[pltpu.VMEM((1, 16), jnp.float32)],
    )(x, Wn1, bn1, Wn2, bn2, Whe_s, Wg2, bg2, Wge_rep, bhe_rep)


def _edge_hop(epk, gpk, ct, Wbd, write_enew):
    nrows = epk.shape[0]
    nsteps = nrows // BE

    def body(e, g_hbm, c, w, *refs):
        if write_enew:
            enew, eres, sume, acc, gbuf, gsem = refs
        else:
            eres, sume, acc, gbuf, gsem = refs
        i = pl.program_id(0)

        @pl.when(i == 0)
        def _():
            acc[...] = jnp.zeros_like(acc)
            pltpu.make_async_copy(
                g_hbm.at[pl.ds(0, BE)], gbuf.at[0], gsem.at[0]
            ).start()

        @pl.when(i + 1 < nsteps)
        def _():
            pltpu.make_async_copy(
                g_hbm.at[pl.ds((i + 1) * BE, BE)],
                gbuf.at[(i + 1) % 2],
                gsem.at[(i + 1) % 2],
            ).start()

        pltpu.make_async_copy(
            g_hbm.at[pl.ds(i * BE, BE)], gbuf.at[i % 2], gsem.at[i % 2]
        ).wait()
        er = jnp.maximum(
            jnp.dot(e[...], w[...], preferred_element_type=jnp.float32)
            + gbuf[i % 2]
            + c[...],
            0.0,
        )
        eres[...] = er
        if write_enew:
            enew[...] = e[...] + er
        acc[...] += jnp.sum(er, axis=0, keepdims=True)

        @pl.when(i == nsteps - 1)
        def _():
            sume[...] = acc[...]

    out_specs = [
        pl.BlockSpec((BE, 128), lambda i: (i, 0)),
        pl.BlockSpec((1, 128), lambda i: (0, 0)),
    ]
    out_shape = [
        jax.ShapeDtypeStruct((nrows, 128), jnp.float32),
        jax.ShapeDtypeStruct((1, 128), jnp.float32),
    ]
    if write_enew:
        out_specs = [pl.BlockSpec((BE, 128), lambda i: (i, 0))] + out_specs
        out_shape = [jax.ShapeDtypeStruct((nrows, 128), jnp.float32)] + out_shape

    return pl.pallas_call(
        body,
        grid=(nsteps,),
        in_specs=[
            pl.BlockSpec((BE, 128), lambda i: (i, 0)),
            pl.BlockSpec(memory_space=pltpu.MemorySpace.HBM),
            pl.BlockSpec((1, 128), lambda i: (0, 0)),
            pl.BlockSpec((128, 128), lambda i: (0, 0)),
        ],
        out_specs=out_specs,
        out_shape=out_shape,
        scratch_shapes=[
            pltpu.VMEM((1, 128), jnp.float32),
            pltpu.VMEM((2, BE, 128), jnp.float32),
            pltpu.SemaphoreType.DMA((2,)),
        ],
        input_output_aliases={1: 1 if write_enew else 0},
    )(epk, gpk, ct, Wbd)


def _node_hop(n, aggA, aggB, cntA, cntB, u, sume, weights, last_hop):
    (Whn_n, Whn_in, Whn_g, bhn, Wge_fold, Whg_n, Whg_g, bhg,
     Whe_s, Wge_rep, bhe_rep, Wro_n, bro_n, Wro_g, bro_g) = weights
    nsteps = N // BN

    def body(nb, aA, aB, cA, cB, ub, se, wnn, wni, wng, cbn, wgef, wgn, wgg,
             cbg, ws, wgr, cbr, wron, bron, wrog, brog, *refs):
        if last_hop:
            nout, gout, acc = refs
        else:
            nnew_r, snext_r, unew_r, cnext_r, acc = refs
        i = pl.program_id(0)

        @pl.when(i == 0)
        def _():
            acc[...] = jnp.zeros_like(acc)

        d = jnp.dot(ub[...], wng[...], preferred_element_type=jnp.float32) + cbn[...]
        agg = (aA[...] + aB[...]) / jnp.maximum(cA[...] + cB[...], 1.0)
        nr = jnp.maximum(
            jnp.dot(nb[...], wnn[...], preferred_element_type=jnp.float32)
            + jnp.dot(agg, wni[...], preferred_element_type=jnp.float32)
            + d,
            0.0,
        )
        nnew = nb[...] + nr
        acc[...] += jnp.sum(nr, axis=0, keepdims=True)
        if last_hop:
            nout[...] = jax.nn.sigmoid(
                jnp.dot(nnew, wron[...], preferred_element_type=jnp.float32)
                + bron[...]
            )
        else:
            nnew_r[...] = nnew
            snext_r[...] = jnp.dot(nnew, ws[...], preferred_element_type=jnp.float32)

        @pl.when(i == nsteps - 1)
        def _():
            ures = jnp.maximum(
                jnp.dot(se[...], wgef[...], preferred_element_type=jnp.float32)
                + jnp.dot(acc[...] / N, wgn[...], preferred_element_type=jnp.float32)
                + jnp.dot(ub[...], wgg[...], preferred_element_type=jnp.float32)
                + cbg[...],
                0.0,
            )
            unew = ub[...] + ures
            if last_hop:
                gout[...] = jax.nn.sigmoid(
                    jnp.dot(unew, wrog[...], preferred_element_type=jnp.float32)
                    + brog[...]
                )
            else:
                unew_r[...] = unew
                cnext_r[...] = (
                    jnp.dot(unew, wgr[...], preferred_element_type=jnp.float32)
                    + cbr[...]
                )

    if last_hop:
        out_specs = [
            pl.BlockSpec((BN, 1), lambda i: (i, 0)),
            pl.BlockSpec((1, 1), lambda i: (0, 0)),
        ]
        out_shape = [
            jax.ShapeDtypeStruct((N, 1), jnp.float32),
            jax.ShapeDtypeStruct((1, 1), jnp.float32),
        ]
    else:
        out_specs = [
            pl.BlockSpec((BN, 16), lambda i: (i, 0)),
            pl.BlockSpec((BN, 8), lambda i: (i, 0)),
            pl.BlockSpec((1, 4), lambda i: (0, 0)),
            pl.BlockSpec((1, 128), lambda i: (0, 0)),
        ]
        out_shape = [
            jax.ShapeDtypeStruct((N, 16), jnp.float32),
            jax.ShapeDtypeStruct((N, 8), jnp.float32),
            jax.ShapeDtypeStruct((1, 4), jnp.float32),
            jax.ShapeDtypeStruct((1, 128), jnp.float32),
        ]

    return pl.pallas_call(
        body,
        grid=(nsteps,),
        in_specs=[
            pl.BlockSpec((BN, 16), lambda i: (i, 0)),
            pl.BlockSpec((BN, 8), lambda i: (i, 0)),
            pl.BlockSpec((BN, 8), lambda i: (i, 0)),
            pl.BlockSpec((BN, 8), lambda i: (i, 0)),
            pl.BlockSpec((BN, 8), lambda i: (i, 0)),
            pl.BlockSpec((1, 4), lambda i: (0, 0)),
            pl.BlockSpec((1, 128), lambda i: (0, 0)),
            pl.BlockSpec((16, 16), lambda i: (0, 0)),
            pl.BlockSpec((8, 16), lambda i: (0, 0)),
            pl.BlockSpec((4, 16), lambda i: (0, 0)),
            pl.BlockSpec((1, 16), lambda i: (0, 0)),
            pl.BlockSpec((128, 4), lambda i: (0, 0)),
            pl.BlockSpec((16, 4), lambda i: (0, 0)),
            pl.BlockSpec((4, 4), lambda i: (0, 0)),
            pl.BlockSpec((1, 4), lambda i: (0, 0)),
            pl.BlockSpec((16, 8), lambda i: (0, 0)),
            pl.BlockSpec((4, 128), lambda i: (0, 0)),
            pl.BlockSpec((1, 128), lambda i: (0, 0)),
            pl.BlockSpec((16, 1), lambda i: (0, 0)),
            pl.BlockSpec((1, 1), lambda i: (0, 0)),
            pl.BlockSpec((4, 1), lambda i: (0, 0)),
            pl.BlockSpec((1, 1), lambda i: (0, 0)),
        ],
        out_specs=out_specs,
        out_shape=out_shape,
        scratch_shapes=[pltpu.VMEM((1, 16), jnp.float32)],
    )(n, aggA, aggB, cntA, cntB, u, sume, Whn_n, Whn_in, Whn_g, bhn,
      Wge_fold, Whg_n, Whg_g, bhg, Whe_s, Wge_rep, bhe_rep, Wro_n, bro_n,
      Wro_g, bro_g)


def kernel(x, edge_attr, senders, receivers, We1, be1, Wn1, bn1, We2, be2,
           Wn2, bn2, Wg2, bg2, Whe_e, Whe_s, Whe_g, bhe, Whn_n, Whn_in,
           Whn_g, bhn, Whg_e, Whg_n, Whg_g, bhg, Wro_n, bro_n, Wro_g, bro_g):
    f32 = jnp.float32
    senders = senders.astype(jnp.int32)
    receivers = receivers.astype(jnp.int32)
    s2d = senders.reshape(NROWS, ROW)
    r2d = receivers.reshape(NROWS, ROW)

    eye16 = jnp.eye(16, dtype=f32)
    W1t = We1.T
    b1c = be1[:, None]
    W2t = We2.T
    b2c = be2[:, None]
    Wbd = jnp.kron(eye16, Whe_e)
    Wge_rep = jnp.tile(Whe_g, (1, 16))
    bhe_rep = jnp.tile(bhe, 16)[None]
    Wge_fold = jnp.tile(Whg_e, (16, 1)) / E
    bn1_2 = bn1[None]
    bn2_2 = bn2[None]
    bg2_2 = bg2[None]
    bhn_2 = bhn[None]
    bhg_2 = bhg[None]
    bro_n2 = bro_n[None]
    bro_g2 = bro_g[None]

    zer = jnp.zeros((N, DE_L), f32)
    ones = jnp.ones((ROW, DE_L), f32)

    e0t = _edge_encoder(edge_attr.T, W1t, b1c, W2t, b2c)
    e0pk = e0t.T.reshape(E // 16, 128)
    n0, s1, u0, c1 = _node_encoder(
        x, Wn1, bn1_2, Wn2, bn2_2, Whe_s, Wg2, bg2_2, Wge_rep, bhe_rep
    )
    cntP = _sc_count(r2d, ones, zer)
    cntA, cntB = cntP[0], cntP[1]

    node_w = (Whn_n, Whn_in, Whn_g, bhn_2, Wge_fold, Whg_n, Whg_g, bhg_2,
              Whe_s, Wge_rep, bhe_rep, Wro_n, bro_n2, Wro_g, bro_g2)

    g1 = _sc_gather(s1, s2d).reshape(E // 16, 128)
    e1pk, er1pk, sume1 = _edge_hop(e0pk, g1, c1, Wbd, write_enew=True)
    aggP1 = _sc_scatter(er1pk.reshape(E, DE_L), r2d, zer)
    n1, s2n, u1, c2 = _node_hop(
        n0, aggP1[0], aggP1[1], cntA, cntB, u0, sume1, node_w, last_hop=False
    )

    g2 = _sc_gather(s2n, s2d).reshape(E // 16, 128)
    er2pk, sume2 = _edge_hop(e1pk, g2, c2, Wbd, write_enew=False)
    aggP2 = _sc_scatter(er2pk.reshape(E, DE_L), r2d, zer)
    nodes_out, globals_out = _node_hop(
        n1, aggP2[0], aggP2[1], cntA, cntB, u1, sume2, node_w, last_hop=True
    )
    return nodes_out, globals_out
